# Initial kernel scaffold; baseline (speedup 1.0000x reference)
import jax
import jax.numpy as jnp
from jax.experimental import pallas as pl


def kernel(x, X, edge_index, a_val, W_comp, b_comp, W1, b1, W2, b2):
    raise NotImplementedError("write your pallas kernel here")


# speedup vs baseline: 3.7283x; 3.7283x over previous
import functools

import jax
import jax.numpy as jnp
from jax import lax
from jax.experimental import pallas as pl
from jax.experimental.pallas import tpu as pltpu
from jax.experimental.pallas import tpu_sc as plsc

NC = 2
NS = 16
LANES = 16
NW = NC * NS

CH = 128
ZROWS = 208
STRIPE = 624


def _spmm_body(n_nodes, feat, support_hbm, src_hbm, dst_hbm, aval_hbm,
               out0_hbm, out1_hbm, src_v, dst_v, av_v, rows_v, zbuf, acc, sem):
    c = lax.axis_index("c")
    s = lax.axis_index("s")
    w = c * NS + s
    n_chunks = src_hbm.shape[0] // CH
    base_trips = n_chunks // NW
    rem = n_chunks % NW
    jg = feat // LANES

    zero = jnp.zeros((LANES,), jnp.float32)

    def zrow(r, carry):
        for j in range(jg):
            zbuf[r, pl.ds(j * LANES, LANES)] = zero
        return carry

    lax.fori_loop(0, ZROWS, zrow, 0)
    row0 = s * STRIPE
    tail0 = NS * STRIPE
    tail_rows = n_nodes - tail0
    for i in range(STRIPE // ZROWS):
        pltpu.sync_copy(zbuf, acc.at[pl.ds(row0 + i * ZROWS, ZROWS), :])
    @pl.when(s == NS - 1)
    def _():
        pltpu.sync_copy(zbuf.at[pl.ds(0, tail_rows), :],
                        acc.at[pl.ds(tail0, tail_rows), :])
    plsc.subcore_barrier()

    start = w * base_trips + jnp.minimum(w, rem)
    trips = base_trips + (w < rem).astype(jnp.int32)

    def chunk(k, carry):
        base = (start + k) * CH
        pltpu.sync_copy(src_hbm.at[pl.ds(base, CH)], src_v)
        pltpu.sync_copy(dst_hbm.at[pl.ds(base, CH)], dst_v)
        pltpu.sync_copy(aval_hbm.at[pl.ds(base, CH)], av_v)
        pltpu.async_copy(support_hbm.at[src_v], rows_v, sem).wait()

        def escale(e, ecarry):
            af = plsc.load_gather(av_v, [jnp.full((LANES,), e, jnp.int32)])
            for j in range(jg):
                sl = pl.ds(j * LANES, LANES)
                rows_v[e, sl] = rows_v[e, sl] * af
            return ecarry

        lax.fori_loop(0, CH, escale, 0)
        pltpu.sync_copy(rows_v, acc.at[dst_v], add=True)
        return carry

    lax.fori_loop(0, trips, chunk, 0)
    plsc.subcore_barrier()

    for i in range(STRIPE // ZROWS):
        r = row0 + i * ZROWS
        @pl.when(c == 0)
        def _():
            pltpu.sync_copy(acc.at[pl.ds(r, ZROWS), :], out0_hbm.at[pl.ds(r, ZROWS), :])
        @pl.when(c == 1)
        def _():
            pltpu.sync_copy(acc.at[pl.ds(r, ZROWS), :], out1_hbm.at[pl.ds(r, ZROWS), :])
    @pl.when(s == NS - 1)
    def _():
        @pl.when(c == 0)
        def _():
            pltpu.sync_copy(acc.at[pl.ds(tail0, tail_rows), :],
                            out0_hbm.at[pl.ds(tail0, tail_rows), :])
        @pl.when(c == 1)
        def _():
            pltpu.sync_copy(acc.at[pl.ds(tail0, tail_rows), :],
                            out1_hbm.at[pl.ds(tail0, tail_rows), :])


def _spmm_partials(support, src, dst, a_val):
    n_nodes, feat = support.shape
    mesh = plsc.VectorSubcoreMesh(core_axis_name="c", subcore_axis_name="s",
                                  num_cores=NC, num_subcores=NS)
    f = pl.kernel(
        functools.partial(_spmm_body, n_nodes, feat),
        out_type=(jax.ShapeDtypeStruct((n_nodes, feat), jnp.float32),
                  jax.ShapeDtypeStruct((n_nodes, feat), jnp.float32)),
        mesh=mesh,
        compiler_params=pltpu.CompilerParams(needs_layout_passes=False),
        scratch_types=[
            pltpu.VMEM((CH,), jnp.int32),
            pltpu.VMEM((CH,), jnp.int32),
            pltpu.VMEM((CH,), jnp.float32),
            pltpu.VMEM((CH, feat), jnp.float32),
            pltpu.VMEM((ZROWS, feat), jnp.float32),
            pltpu.VMEM_SHARED((n_nodes, feat), jnp.float32),
            pltpu.SemaphoreType.DMA,
        ],
    )
    return f(support, src, dst, a_val)


def _k1_body(X_ref, Wc_ref, W1_ref, bc_ref, out_ref, wf_s, bf_s):
    @pl.when(pl.program_id(0) == 0)
    def _():
        wf_s[...] = jnp.dot(Wc_ref[...], W1_ref[...],
                            preferred_element_type=jnp.float32)
        bf_s[...] = jnp.dot(bc_ref[...], W1_ref[...],
                            preferred_element_type=jnp.float32)
    out_ref[...] = jnp.dot(X_ref[...], wf_s[...],
                           preferred_element_type=jnp.float32) + bf_s[...]


def _k1(X, W_comp, W1, b_comp):
    n, d = X.shape
    hid = W1.shape[1]
    bm = 1000
    return pl.pallas_call(
        _k1_body,
        grid=(n // bm,),
        in_specs=[
            pl.BlockSpec((bm, d), lambda i: (i, 0)),
            pl.BlockSpec(W_comp.shape, lambda i: (0, 0)),
            pl.BlockSpec(W1.shape, lambda i: (0, 0)),
            pl.BlockSpec((1, W1.shape[0]), lambda i: (0, 0)),
        ],
        out_specs=pl.BlockSpec((bm, hid), lambda i: (i, 0)),
        out_shape=jax.ShapeDtypeStruct((n, hid), jnp.float32),
        scratch_shapes=[
            pltpu.VMEM((d, hid), jnp.float32),
            pltpu.VMEM((1, hid), jnp.float32),
        ],
    )(X, W_comp, W1, b_comp.reshape(1, -1))


def _k2_body(za_ref, zb_ref, b1_ref, W2_ref, out_ref):
    f1 = jnp.maximum(za_ref[...] + zb_ref[...] + b1_ref[...], 0.0)
    out_ref[...] = jnp.dot(f1, W2_ref[...], preferred_element_type=jnp.float32)


def _k2(za, zb, b1, W2):
    n, hid = za.shape
    out_c = W2.shape[1]
    bm = 1000
    return pl.pallas_call(
        _k2_body,
        grid=(n // bm,),
        in_specs=[
            pl.BlockSpec((bm, hid), lambda i: (i, 0)),
            pl.BlockSpec((bm, hid), lambda i: (i, 0)),
            pl.BlockSpec((1, hid), lambda i: (0, 0)),
            pl.BlockSpec(W2.shape, lambda i: (0, 0)),
        ],
        out_specs=pl.BlockSpec((bm, out_c), lambda i: (i, 0)),
        out_shape=jax.ShapeDtypeStruct((n, out_c), jnp.float32),
    )(za, zb, b1.reshape(1, -1), W2)


def _k3_body(x_ref, za_ref, zb_ref, b2_ref, out_ref):
    f2 = za_ref[...] + zb_ref[...] + b2_ref[...]
    acc = lax.dot_general(x_ref[...], f2, (((1,), (1,)), ((), ())),
                          preferred_element_type=jnp.float32)
    out_ref[...] = jax.nn.sigmoid(acc)


def _k3(x, za, zb, b2):
    b, out_c = x.shape
    n = za.shape[0]
    bm = 128
    return pl.pallas_call(
        _k3_body,
        grid=(b // bm,),
        in_specs=[
            pl.BlockSpec((bm, out_c), lambda i: (i, 0)),
            pl.BlockSpec((n, out_c), lambda i: (0, 0)),
            pl.BlockSpec((n, out_c), lambda i: (0, 0)),
            pl.BlockSpec((1, out_c), lambda i: (0, 0)),
        ],
        out_specs=pl.BlockSpec((bm, n), lambda i: (i, 0)),
        out_shape=jax.ShapeDtypeStruct((b, n), jnp.float32),
    )(x, za, zb, b2.reshape(1, -1))


def kernel(x, X, edge_index, a_val, W_comp, b_comp, W1, b1, W2, b2):
    src = edge_index[0]
    dst = edge_index[1]
    support1 = _k1(X, W_comp, W1, b_comp)
    z1a, z1b = _spmm_partials(support1, src, dst, a_val)
    support2 = _k2(z1a, z1b, b1, W2)
    z2a, z2b = _spmm_partials(support2, src, dst, a_val)
    return _k3(x, z2a, z2b, b2)
